# baseline (speedup 1.0000x reference)
import jax
import jax.numpy as jnp
from jax.experimental import pallas as pl

_R = 8


def _body(tok_ref, feat_ref, out_ref, *, hw, c, topk):
    tok = tok_ref[...]
    feat = feat_ref[...]
    scale = c ** -0.5
    feat2d = feat.reshape(hw * feat.shape[0], c)
    s_full = jax.lax.dot_general(
        feat2d, tok, (((1,), (1,)), ((), ())),
        precision=jax.lax.Precision.DEFAULT,
        preferred_element_type=jnp.float32)
    s3 = s_full.reshape(feat.shape[0], hw, feat.shape[0])
    rr = jax.lax.broadcasted_iota(jnp.int32, s3.shape, 0)
    ll = jax.lax.broadcasted_iota(jnp.int32, s3.shape, 2)
    s = jnp.sum(jnp.where(rr == ll, s3, 0.0), axis=-1) * scale
    m = jnp.max(s, axis=-1, keepdims=True)

    bits = jax.lax.bitcast_convert_type(s, jnp.int32)
    key = jnp.where(bits >= 0, bits, bits ^ jnp.int32(0x7FFFFFFF))
    lo = jnp.min(key, axis=-1, keepdims=True)
    hi = jnp.max(key, axis=-1, keepdims=True) + 1
    for _ in range(32):
        mid = (lo >> 1) + (hi >> 1) + (lo & hi & 1)
        cnt = jnp.sum((key >= mid).astype(jnp.int32), axis=-1,
                      keepdims=True)
        ge = cnt >= topk
        lo = jnp.where(ge, mid, lo)
        hi = jnp.where(ge, hi, mid)
    w = jnp.where(key >= lo, jnp.exp(s - m), 0.0)
    denom = jnp.sum(w, axis=-1, keepdims=True)
    out = jnp.sum(feat * w[:, :, None], axis=1)
    out_ref[...] = out / denom


def kernel(token, feature):
    b, n, k, c = token.shape
    hw = feature.shape[3]
    nrows = b * n * k
    topk = int(hw * 0.5)
    tok = token.reshape(nrows, c)
    feat = feature.reshape(nrows, hw, c)

    import functools
    body = functools.partial(_body, hw=hw, c=c, topk=topk)
    out = pl.pallas_call(
        body,
        grid=(nrows // _R,),
        in_specs=[
            pl.BlockSpec((_R, c), lambda i: (i, 0)),
            pl.BlockSpec((_R, hw, c), lambda i: (i, 0, 0)),
        ],
        out_specs=pl.BlockSpec((_R, c), lambda i: (i, 0)),
        out_shape=jax.ShapeDtypeStruct((nrows, c), jnp.float32),
    )(tok, feat)
    return out.reshape(b, n, k, c)


# speedup vs baseline: 4.7333x; 4.7333x over previous
import jax
import jax.numpy as jnp
from jax.experimental import pallas as pl

_R = 8


def _body(tok_ref, feat_ref, out_ref, *, hw, c, topk):
    tok = tok_ref[...]
    feat = feat_ref[...]
    scale = c ** -0.5
    feat2d = feat.reshape(hw * feat.shape[0], c)
    s_full = jax.lax.dot_general(
        feat2d, tok, (((1,), (1,)), ((), ())),
        precision=jax.lax.Precision.DEFAULT,
        preferred_element_type=jnp.float32)
    s3 = s_full.reshape(feat.shape[0], hw, feat.shape[0])
    rr = jax.lax.broadcasted_iota(jnp.int32, s3.shape, 0)
    ll = jax.lax.broadcasted_iota(jnp.int32, s3.shape, 2)
    s = jnp.sum(jnp.where(rr == ll, s3, 0.0), axis=-1) * scale
    m = jnp.max(s, axis=-1, keepdims=True)

    bits = jax.lax.bitcast_convert_type(s, jnp.int32)
    key = jnp.where(bits >= 0, bits, bits ^ jnp.int32(0x7FFFFFFF))
    keyT = jnp.transpose(key)
    lo = jnp.min(keyT, axis=0, keepdims=True)
    hi = jnp.max(keyT, axis=0, keepdims=True) + 1
    for _ in range(32):
        mid = (lo >> 1) + (hi >> 1) + (lo & hi & 1)
        cnt = jnp.sum((keyT >= mid).astype(jnp.int32), axis=0,
                      keepdims=True)
        ge = cnt >= topk
        lo = jnp.where(ge, mid, lo)
        hi = jnp.where(ge, hi, mid)
    t = jnp.transpose(lo)
    w = jnp.where(key >= t, jnp.exp(s - m), 0.0)
    denom = jnp.sum(w, axis=-1, keepdims=True)
    out = jnp.sum(feat * w[:, :, None], axis=1)
    out_ref[...] = out / denom


def kernel(token, feature):
    b, n, k, c = token.shape
    hw = feature.shape[3]
    nrows = b * n * k
    topk = int(hw * 0.5)
    tok = token.reshape(nrows, c)
    feat = feature.reshape(nrows, hw, c)

    import functools
    body = functools.partial(_body, hw=hw, c=c, topk=topk)
    out = pl.pallas_call(
        body,
        grid=(nrows // _R,),
        in_specs=[
            pl.BlockSpec((_R, c), lambda i: (i, 0)),
            pl.BlockSpec((_R, hw, c), lambda i: (i, 0, 0)),
        ],
        out_specs=pl.BlockSpec((_R, c), lambda i: (i, 0)),
        out_shape=jax.ShapeDtypeStruct((nrows, c), jnp.float32),
    )(tok, feat)
    return out.reshape(b, n, k, c)


# speedup vs baseline: 8.8604x; 1.8719x over previous
import functools

import jax
import jax.numpy as jnp
from jax.experimental import pallas as pl

_R = 32
_RC = 8


def _body(tok_ref, feat_ref, out_ref, *, hw, c, topk):
    tok = tok_ref[...]
    feat = feat_ref[...]
    scale = c ** -0.5

    parts = []
    for q in range(_R // _RC):
        fq = feat[q * _RC:(q + 1) * _RC]
        tq = tok[q * _RC:(q + 1) * _RC]
        f2 = fq.reshape(_RC * hw, c)
        sf = jax.lax.dot_general(
            f2, tq, (((1,), (1,)), ((), ())),
            precision=jax.lax.Precision.DEFAULT,
            preferred_element_type=jnp.float32)
        s3 = sf.reshape(_RC, hw, _RC)
        rr = jax.lax.broadcasted_iota(jnp.int32, s3.shape, 0)
        ll = jax.lax.broadcasted_iota(jnp.int32, s3.shape, 2)
        parts.append(jnp.sum(jnp.where(rr == ll, s3, 0.0), axis=-1))
    s = jnp.concatenate(parts, axis=0) * scale
    m = jnp.max(s, axis=-1, keepdims=True)

    bits = jax.lax.bitcast_convert_type(s, jnp.int32)
    key = jnp.where(bits >= 0, bits, bits ^ jnp.int32(0x7FFFFFFF))
    keyT = jnp.transpose(key)
    lo = jnp.min(keyT, axis=0, keepdims=True)
    hi = jnp.max(keyT, axis=0, keepdims=True) + 1
    for _ in range(32):
        mid = (lo >> 1) + (hi >> 1) + (lo & hi & 1)
        cnt = jnp.sum((keyT >= mid).astype(jnp.int32), axis=0,
                      keepdims=True)
        ge = cnt >= topk
        lo = jnp.where(ge, mid, lo)
        hi = jnp.where(ge, hi, mid)
    t = jnp.transpose(lo)

    w = jnp.where(key >= t, jnp.exp(s - m), 0.0)
    denom = jnp.sum(w, axis=-1, keepdims=True)
    out = jnp.sum(feat * w[:, :, None], axis=1)
    out_ref[...] = out / denom


def kernel(token, feature):
    b, n, k, c = token.shape
    hw = feature.shape[3]
    nrows = b * n * k
    topk = int(hw * 0.5)
    tok = token.reshape(nrows, c)
    feat = feature.reshape(nrows, hw, c)

    body = functools.partial(_body, hw=hw, c=c, topk=topk)
    out = pl.pallas_call(
        body,
        grid=(nrows // _R,),
        in_specs=[
            pl.BlockSpec((_R, c), lambda i: (i, 0)),
            pl.BlockSpec((_R, hw, c), lambda i: (i, 0, 0)),
        ],
        out_specs=pl.BlockSpec((_R, c), lambda i: (i, 0)),
        out_shape=jax.ShapeDtypeStruct((nrows, c), jnp.float32),
    )(tok, feat)
    return out.reshape(b, n, k, c)
